# baseline (speedup 1.0000x reference)
import dataclasses
import functools

import jax
import jax.numpy as jnp
from jax import lax
from jax.experimental import pallas as pl
from jax.experimental.pallas import tpu as pltpu
from jax.experimental.pallas import tpu_sc as plsc

TILE = 256
TOK_BLK = 512
NW = 32
LANES = 16


def _sigmoid(x):
    return 1.0 / (1.0 + jnp.exp(-x))


def _sc_compiler_params():
    cp = pltpu.CompilerParams()
    if "needs_layout_passes" in pltpu.CompilerParams.__dataclass_fields__:
        cp = dataclasses.replace(cp, needs_layout_passes=False)
    return cp


def _router_body(x_ref, rw_ref, xs_ref, sc_ref, eid_ref, grk_ref, cnt_ref,
                 meta_ref, counts):
    i = pl.program_id(0)
    nblk = pl.num_programs(0)
    num_e = rw_ref.shape[1]

    @pl.when(i == 0)
    def _():
        counts[...] = jnp.zeros_like(counts)

    x = x_ref[...]
    logits = jnp.dot(x, rw_ref[...], preferred_element_type=jnp.float32)
    m = jnp.max(logits, axis=1, keepdims=True)
    iota_e = lax.broadcasted_iota(jnp.int32, logits.shape, 1)
    eid = jnp.min(jnp.where(logits == m, iota_e, num_e), axis=1,
                  keepdims=True)
    onehot = (iota_e == eid).astype(jnp.float32)
    s = _sigmoid(m)
    xs_ref[...] = (x * s).astype(xs_ref.dtype)

    sc_ref[...] = _sigmoid(jnp.where(onehot > 0, logits, -jnp.inf))

    ii = lax.broadcasted_iota(jnp.int32, (TOK_BLK, TOK_BLK), 0)
    jj = lax.broadcasted_iota(jnp.int32, (TOK_BLK, TOK_BLK), 1)
    tril = (ii > jj).astype(jnp.float32)

    prefix = jnp.dot(tril, onehot, preferred_element_type=jnp.float32)
    local = jnp.sum(prefix * onehot, axis=1, keepdims=True)
    carried = jnp.sum(onehot * counts[...], axis=1, keepdims=True)
    grank = local + carried
    counts[...] = counts[...] + jnp.sum(onehot, axis=0, keepdims=True)

    eid_ref[...] = eid.reshape(1, TOK_BLK, 1)
    grk_ref[...] = grank.astype(jnp.int32).reshape(1, TOK_BLK, 1)

    @pl.when(i == nblk - 1)
    def _():
        c16 = jnp.concatenate([counts[...], jnp.zeros_like(counts)], axis=1)
        cnt_ref[...] = c16.astype(jnp.int32).reshape(1, 1, 2 * num_e)

        pc16 = jnp.ceil(c16 * (1.0 / TILE)) * TILE
        u16i = lax.broadcasted_iota(jnp.int32, (16, 16), 0)
        u16j = lax.broadcasted_iota(jnp.int32, (16, 16), 1)
        u16 = (u16i <= u16j).astype(jnp.float32)
        incl = jnp.dot(jnp.broadcast_to(pc16, (8, 16)), u16,
                       preferred_element_type=jnp.float32)[0:1]
        total = jnp.max(incl, keepdims=True)
        last = total * (1.0 / TILE) - 1.0
        nt2 = meta_ref.shape[1]
        itile = lax.broadcasted_iota(jnp.int32, (nt2, 1), 0).astype(jnp.float32)
        ic_col = jnp.minimum(itile, last)
        start_col = ic_col * TILE
        cmp = (incl <= start_col).astype(jnp.float32)
        te_col = jnp.sum(cmp, axis=1, keepdims=True)
        ii2 = lax.broadcasted_iota(jnp.int32, (nt2, nt2), 0)
        jj2 = lax.broadcasted_iota(jnp.int32, (nt2, nt2), 1)
        ident2 = (ii2 == jj2).astype(jnp.float32)

        def dot_t2(a):
            return lax.dot_general(a, ident2, (((0,), (0,)), ((), ())),
                                   preferred_element_type=jnp.float32)

        meta = jnp.concatenate(
            [dot_t2(te_col), dot_t2(ic_col),
             jnp.zeros((meta_ref.shape[0] - 2, nt2), jnp.float32)], axis=0)
        meta_ref[...] = meta.astype(jnp.int32)


def _router(x, rw):
    t, d = x.shape
    nblk = t // TOK_BLK
    num_e = rw.shape[1]
    return pl.pallas_call(
        _router_body,
        grid=(nblk,),
        in_specs=[
            pl.BlockSpec((TOK_BLK, d), lambda i: (i, 0)),
            pl.BlockSpec((d, num_e), lambda i: (0, 0)),
        ],
        out_specs=[
            pl.BlockSpec((TOK_BLK, d), lambda i: (i, 0)),
            pl.BlockSpec((TOK_BLK, num_e), lambda i: (i, 0)),
            pl.BlockSpec((1, TOK_BLK, 1), lambda i: (i, 0, 0)),
            pl.BlockSpec((1, TOK_BLK, 1), lambda i: (i, 0, 0)),
            pl.BlockSpec((1, 1, 2 * num_e), lambda i: (0, 0, 0)),
            pl.BlockSpec((8, 2 * LANES), lambda i: (0, 0)),
        ],
        out_shape=[
            jax.ShapeDtypeStruct((t, d), jnp.float32),
            jax.ShapeDtypeStruct((t, num_e), jnp.float32),
            jax.ShapeDtypeStruct((nblk, TOK_BLK, 1), jnp.int32),
            jax.ShapeDtypeStruct((nblk, TOK_BLK, 1), jnp.int32),
            jax.ShapeDtypeStruct((1, 1, 2 * num_e), jnp.int32),
            jax.ShapeDtypeStruct((8, 2 * LANES), jnp.int32),
        ],
        scratch_shapes=[pltpu.VMEM((1, num_e), jnp.float32)],
    )(x, rw)


def _dispatch(xs, eid, grank, counts, nt):
    t, d = xs.shape
    bpw = t // NW
    num_e = 8
    mesh = plsc.VectorSubcoreMesh(core_axis_name="c", subcore_axis_name="s")

    @functools.partial(
        pl.kernel,
        mesh=mesh,
        compiler_params=_sc_compiler_params(),
        out_type=[
            jax.ShapeDtypeStruct((nt * TILE, d), jnp.float32),
            jax.ShapeDtypeStruct((t,), jnp.int32),
        ],
        scratch_types=[
            pltpu.VMEM((bpw,), jnp.int32),
            pltpu.VMEM((bpw,), jnp.int32),
            pltpu.VMEM((1, bpw), jnp.int32),
            pltpu.VMEM((LANES,), jnp.int32),
            pltpu.VMEM((LANES,), jnp.int32),
            pltpu.VMEM((bpw, d), jnp.float32),
        ],
    )
    def k(xs_hbm, eid_hbm, grk_hbm, cnt_hbm, y_hbm, pos_hbm,
          eid_v, grk_v, pos_v, cnt_v, off_v, rows_v):
        wid = lax.axis_index("s") * 2 + lax.axis_index("c")
        base = wid * bpw
        pltpu.sync_copy(eid_hbm.at[pl.ds(base, bpw)], eid_v)
        pltpu.sync_copy(grk_hbm.at[pl.ds(base, bpw)], grk_v)
        pltpu.sync_copy(cnt_hbm, cnt_v)
        c = cnt_v[...]
        pc = (c + (TILE - 1)) & (-TILE)
        incl = plsc.cumsum(pc)
        off_v[...] = incl - pc
        for j in range(bpw // LANES):
            ev = eid_v[pl.ds(LANES * j, LANES)]
            gv = grk_v[pl.ds(LANES * j, LANES)]
            ov = plsc.load_gather(off_v, [ev])
            pos_v[0, pl.ds(LANES * j, LANES)] = ov + gv
        pltpu.sync_copy(pos_v.at[0], pos_hbm.at[pl.ds(base, bpw)])
        pltpu.sync_copy(xs_hbm.at[pl.ds(base, bpw)], rows_v)
        pltpu.sync_copy(rows_v, y_hbm.at[pos_v.at[0]])

    return k(xs, eid, grank, counts)


def _grouped(meta, y, wg, wu, wdn, nt):
    _, d, f = wg.shape

    def body(meta_ref, y_ref, wg_ref, wu_ref, wdn_ref, o_ref):
        i = pl.program_id(0)

        @pl.when(i == meta_ref[1, i])
        def _():
            yb = y_ref[...].astype(jnp.float32)
            g = jnp.dot(yb, wg_ref[0], preferred_element_type=jnp.float32)
            u = jnp.dot(yb, wu_ref[0], preferred_element_type=jnp.float32)
            h = g * _sigmoid(g) * u
            o_ref[...] = jnp.dot(h, wdn_ref[0],
                                 preferred_element_type=jnp.float32
                                 ).astype(o_ref.dtype)

    grid_spec = pltpu.PrefetchScalarGridSpec(
        num_scalar_prefetch=1,
        grid=(nt,),
        in_specs=[
            pl.BlockSpec((TILE, d), lambda i, m: (m[1, i], 0)),
            pl.BlockSpec((1, d, f), lambda i, m: (m[0, i], 0, 0)),
            pl.BlockSpec((1, d, f), lambda i, m: (m[0, i], 0, 0)),
            pl.BlockSpec((1, f, d), lambda i, m: (m[0, i], 0, 0)),
        ],
        out_specs=pl.BlockSpec((TILE, d), lambda i, m: (m[1, i], 0)),
    )
    return pl.pallas_call(
        body,
        grid_spec=grid_spec,
        out_shape=jax.ShapeDtypeStruct((nt * TILE, d), jnp.float32),
    )(meta, y, wg, wu, wdn)


def _combine(osort, pos):
    t = pos.shape[0]
    d = osort.shape[1]
    bpw = t // NW
    mesh = plsc.VectorSubcoreMesh(core_axis_name="c", subcore_axis_name="s")

    @functools.partial(
        pl.kernel,
        mesh=mesh,
        compiler_params=_sc_compiler_params(),
        out_type=jax.ShapeDtypeStruct((t, d), jnp.float32),
        scratch_types=[
            pltpu.VMEM((bpw,), jnp.int32),
            pltpu.VMEM((bpw, d), jnp.float32),
            pltpu.SemaphoreType.DMA,
        ],
    )
    def k(os_hbm, pos_hbm, r_hbm, pos_v, rows_v, sem):
        wid = lax.axis_index("s") * 2 + lax.axis_index("c")
        base = wid * bpw
        pltpu.sync_copy(pos_hbm.at[pl.ds(base, bpw)], pos_v)
        pltpu.async_copy(os_hbm.at[pos_v], rows_v, sem).wait()
        pltpu.sync_copy(rows_v, r_hbm.at[pl.ds(base, bpw)])

    return k(osort, pos)


def _shared(x, wsg, wsu, wsdn):
    t, d = x.shape
    blk = 256

    def body(x_ref, g_ref, u_ref, dn_ref, o_ref):
        xb = x_ref[...].astype(jnp.bfloat16)
        g = jnp.dot(xb, g_ref[...], preferred_element_type=jnp.float32)
        u = jnp.dot(xb, u_ref[...], preferred_element_type=jnp.float32)
        h = (g * _sigmoid(g) * u).astype(jnp.bfloat16)
        o_ref[...] = jnp.dot(h, dn_ref[...], preferred_element_type=jnp.float32)

    return pl.pallas_call(
        body,
        grid=(t // blk,),
        in_specs=[
            pl.BlockSpec((blk, d), lambda i: (i, 0)),
            pl.BlockSpec(wsg.shape, lambda i: (0, 0)),
            pl.BlockSpec(wsu.shape, lambda i: (0, 0)),
            pl.BlockSpec(wsdn.shape, lambda i: (0, 0)),
        ],
        out_specs=pl.BlockSpec((blk, d), lambda i: (i, 0)),
        out_shape=jax.ShapeDtypeStruct((t, d), jnp.float32),
    )(x, wsg, wsu, wsdn)


def _add(a, b):
    t, d = a.shape
    blk = 512

    def body(a_ref, b_ref, o_ref):
        o_ref[...] = a_ref[...] + b_ref[...].astype(jnp.float32)

    return pl.pallas_call(
        body,
        grid=(t // blk,),
        in_specs=[
            pl.BlockSpec((blk, a.shape[1]), lambda i: (i, 0)),
            pl.BlockSpec((blk, a.shape[1]), lambda i: (i, 0)),
        ],
        out_specs=pl.BlockSpec((blk, d), lambda i: (i, 0)),
        out_shape=jax.ShapeDtypeStruct((t, d), jnp.float32),
    )(a, b)


def kernel(hidden_states, router_w, gate_proj, up_proj, down_proj,
           shared_gate, shared_up, shared_down):
    b, s, d = hidden_states.shape
    t = b * s
    num_e = router_w.shape[1]
    nt = t // TILE + num_e

    x = hidden_states.reshape(t, d)

    xs, scores_te, eid3, grk3, cnt3, meta = _router(x, router_w)
    eid = eid3.reshape(t)
    grk = grk3.reshape(t)
    cnt = cnt3.reshape(2 * num_e)

    y, pos = _dispatch(xs, eid, grk, cnt, nt)
    osort = _grouped(meta, y, gate_proj, up_proj, down_proj, nt)
    routed = _combine(osort, pos)

    shared = _shared(x, shared_gate.astype(jnp.bfloat16),
                     shared_up.astype(jnp.bfloat16),
                     shared_down.astype(jnp.bfloat16))
    out = _add(shared, routed)
    return out, scores_te.T


# speedup vs baseline: 1.0411x; 1.0411x over previous
import dataclasses
import functools

import jax
import jax.numpy as jnp
from jax import lax
from jax.experimental import pallas as pl
from jax.experimental.pallas import tpu as pltpu
from jax.experimental.pallas import tpu_sc as plsc

TILE = 256
TOK_BLK = 512
NW = 32
LANES = 16


def _sigmoid(x):
    return 1.0 / (1.0 + jnp.exp(-x))


def _sc_compiler_params():
    cp = pltpu.CompilerParams()
    if "needs_layout_passes" in pltpu.CompilerParams.__dataclass_fields__:
        cp = dataclasses.replace(cp, needs_layout_passes=False)
    return cp


def _router_body(x_ref, rw_ref, xs_ref, sc_ref, eid_ref, grk_ref, cnt_ref,
                 meta_ref, counts):
    i = pl.program_id(0)
    nblk = pl.num_programs(0)
    num_e = rw_ref.shape[1]

    @pl.when(i == 0)
    def _():
        counts[...] = jnp.zeros_like(counts)

    x = x_ref[...]
    logits = jnp.dot(x, rw_ref[...], preferred_element_type=jnp.float32)
    m = jnp.max(logits, axis=1, keepdims=True)
    iota_e = lax.broadcasted_iota(jnp.int32, logits.shape, 1)
    eid = jnp.min(jnp.where(logits == m, iota_e, num_e), axis=1,
                  keepdims=True)
    onehot = (iota_e == eid).astype(jnp.float32)
    s = _sigmoid(m)
    xs_ref[...] = (x * s).astype(xs_ref.dtype)

    sc_ref[...] = _sigmoid(jnp.where(onehot > 0, logits, -jnp.inf))

    ii = lax.broadcasted_iota(jnp.int32, (TOK_BLK, TOK_BLK), 0)
    jj = lax.broadcasted_iota(jnp.int32, (TOK_BLK, TOK_BLK), 1)
    tril = (ii > jj).astype(jnp.float32)

    prefix = jnp.dot(tril, onehot, preferred_element_type=jnp.float32)
    local = jnp.sum(prefix * onehot, axis=1, keepdims=True)
    carried = jnp.sum(onehot * counts[...], axis=1, keepdims=True)
    grank = local + carried
    counts[...] = counts[...] + jnp.sum(onehot, axis=0, keepdims=True)

    eid_ref[...] = eid.reshape(1, TOK_BLK, 1)
    grk_ref[...] = grank.astype(jnp.int32).reshape(1, TOK_BLK, 1)

    @pl.when(i == nblk - 1)
    def _():
        c16 = jnp.concatenate([counts[...], jnp.zeros_like(counts)], axis=1)
        cnt_ref[...] = c16.astype(jnp.int32).reshape(1, 1, 2 * num_e)

        pc16 = jnp.ceil(c16 * (1.0 / TILE)) * TILE
        u16i = lax.broadcasted_iota(jnp.int32, (16, 16), 0)
        u16j = lax.broadcasted_iota(jnp.int32, (16, 16), 1)
        u16 = (u16i <= u16j).astype(jnp.float32)
        incl = jnp.dot(jnp.broadcast_to(pc16, (8, 16)), u16,
                       preferred_element_type=jnp.float32)[0:1]
        total = jnp.max(incl, keepdims=True)
        last = total * (1.0 / TILE) - 1.0
        nt2 = meta_ref.shape[1]
        itile = lax.broadcasted_iota(jnp.int32, (nt2, 1), 0).astype(jnp.float32)
        ic_col = jnp.minimum(itile, last)
        start_col = ic_col * TILE
        cmp = (incl <= start_col).astype(jnp.float32)
        te_col = jnp.sum(cmp, axis=1, keepdims=True)
        ii2 = lax.broadcasted_iota(jnp.int32, (nt2, nt2), 0)
        jj2 = lax.broadcasted_iota(jnp.int32, (nt2, nt2), 1)
        ident2 = (ii2 == jj2).astype(jnp.float32)

        def dot_t2(a):
            return lax.dot_general(a, ident2, (((0,), (0,)), ((), ())),
                                   preferred_element_type=jnp.float32)

        meta = jnp.concatenate(
            [dot_t2(te_col), dot_t2(ic_col),
             jnp.zeros((meta_ref.shape[0] - 2, nt2), jnp.float32)], axis=0)
        meta_ref[...] = meta.astype(jnp.int32)


def _router(x, rw):
    t, d = x.shape
    nblk = t // TOK_BLK
    num_e = rw.shape[1]
    return pl.pallas_call(
        _router_body,
        grid=(nblk,),
        in_specs=[
            pl.BlockSpec((TOK_BLK, d), lambda i: (i, 0)),
            pl.BlockSpec((d, num_e), lambda i: (0, 0)),
        ],
        out_specs=[
            pl.BlockSpec((TOK_BLK, d), lambda i: (i, 0)),
            pl.BlockSpec((TOK_BLK, num_e), lambda i: (i, 0)),
            pl.BlockSpec((1, TOK_BLK, 1), lambda i: (i, 0, 0)),
            pl.BlockSpec((1, TOK_BLK, 1), lambda i: (i, 0, 0)),
            pl.BlockSpec((1, 1, 2 * num_e), lambda i: (0, 0, 0)),
            pl.BlockSpec((8, 2 * LANES), lambda i: (0, 0)),
        ],
        out_shape=[
            jax.ShapeDtypeStruct((t, d), jnp.float32),
            jax.ShapeDtypeStruct((t, num_e), jnp.float32),
            jax.ShapeDtypeStruct((nblk, TOK_BLK, 1), jnp.int32),
            jax.ShapeDtypeStruct((nblk, TOK_BLK, 1), jnp.int32),
            jax.ShapeDtypeStruct((1, 1, 2 * num_e), jnp.int32),
            jax.ShapeDtypeStruct((8, 2 * LANES), jnp.int32),
        ],
        scratch_shapes=[pltpu.VMEM((1, num_e), jnp.float32)],
    )(x, rw)


def _dispatch(xs, eid, grank, counts, nt):
    t, d = xs.shape
    bpw = t // NW
    num_e = 8
    mesh = plsc.VectorSubcoreMesh(core_axis_name="c", subcore_axis_name="s")

    @functools.partial(
        pl.kernel,
        mesh=mesh,
        compiler_params=_sc_compiler_params(),
        out_type=[
            jax.ShapeDtypeStruct((nt * TILE, d), jnp.float32),
            jax.ShapeDtypeStruct((t,), jnp.int32),
        ],
        scratch_types=[
            pltpu.VMEM((bpw,), jnp.int32),
            pltpu.VMEM((bpw,), jnp.int32),
            pltpu.VMEM((1, bpw), jnp.int32),
            pltpu.VMEM((LANES,), jnp.int32),
            pltpu.VMEM((LANES,), jnp.int32),
            pltpu.VMEM((bpw, d), jnp.float32),
        ],
    )
    def k(xs_hbm, eid_hbm, grk_hbm, cnt_hbm, y_hbm, pos_hbm,
          eid_v, grk_v, pos_v, cnt_v, off_v, rows_v):
        wid = lax.axis_index("s") * 2 + lax.axis_index("c")
        base = wid * bpw
        pltpu.sync_copy(eid_hbm.at[pl.ds(base, bpw)], eid_v)
        pltpu.sync_copy(grk_hbm.at[pl.ds(base, bpw)], grk_v)
        pltpu.sync_copy(cnt_hbm, cnt_v)
        c = cnt_v[...]
        pc = (c + (TILE - 1)) & (-TILE)
        incl = plsc.cumsum(pc)
        off_v[...] = incl - pc
        for j in range(bpw // LANES):
            ev = eid_v[pl.ds(LANES * j, LANES)]
            gv = grk_v[pl.ds(LANES * j, LANES)]
            ov = plsc.load_gather(off_v, [ev])
            pos_v[0, pl.ds(LANES * j, LANES)] = ov + gv
        pltpu.sync_copy(pos_v.at[0], pos_hbm.at[pl.ds(base, bpw)])
        pltpu.sync_copy(xs_hbm.at[pl.ds(base, bpw)], rows_v)
        pltpu.sync_copy(rows_v, y_hbm.at[pos_v.at[0]])

    return k(xs, eid, grank, counts)


def _grouped(meta, y, wg, wu, wdn, nt):
    _, d, f = wg.shape

    def body(meta_ref, y_ref, wg_ref, wu_ref, wdn_ref, o_ref):
        i = pl.program_id(0)

        @pl.when(i == meta_ref[1, i])
        def _():
            yb = y_ref[...].astype(jnp.bfloat16)
            g = jnp.dot(yb, wg_ref[0].astype(jnp.bfloat16),
                        preferred_element_type=jnp.float32)
            u = jnp.dot(yb, wu_ref[0].astype(jnp.bfloat16),
                        preferred_element_type=jnp.float32)
            h = (g * _sigmoid(g) * u).astype(jnp.bfloat16)
            o_ref[...] = jnp.dot(h, wdn_ref[0].astype(jnp.bfloat16),
                                 preferred_element_type=jnp.float32
                                 ).astype(o_ref.dtype)

    grid_spec = pltpu.PrefetchScalarGridSpec(
        num_scalar_prefetch=1,
        grid=(nt,),
        in_specs=[
            pl.BlockSpec((TILE, d), lambda i, m: (m[1, i], 0)),
            pl.BlockSpec((1, d, f), lambda i, m: (m[0, i], 0, 0)),
            pl.BlockSpec((1, d, f), lambda i, m: (m[0, i], 0, 0)),
            pl.BlockSpec((1, f, d), lambda i, m: (m[0, i], 0, 0)),
        ],
        out_specs=pl.BlockSpec((TILE, d), lambda i, m: (m[1, i], 0)),
    )
    return pl.pallas_call(
        body,
        grid_spec=grid_spec,
        out_shape=jax.ShapeDtypeStruct((nt * TILE, d), jnp.float32),
    )(meta, y, wg, wu, wdn)


def _combine(osort, pos):
    t = pos.shape[0]
    d = osort.shape[1]
    bpw = t // NW
    mesh = plsc.VectorSubcoreMesh(core_axis_name="c", subcore_axis_name="s")

    @functools.partial(
        pl.kernel,
        mesh=mesh,
        compiler_params=_sc_compiler_params(),
        out_type=jax.ShapeDtypeStruct((t, d), jnp.float32),
        scratch_types=[
            pltpu.VMEM((bpw,), jnp.int32),
            pltpu.VMEM((bpw, d), jnp.float32),
            pltpu.SemaphoreType.DMA,
        ],
    )
    def k(os_hbm, pos_hbm, r_hbm, pos_v, rows_v, sem):
        wid = lax.axis_index("s") * 2 + lax.axis_index("c")
        base = wid * bpw
        pltpu.sync_copy(pos_hbm.at[pl.ds(base, bpw)], pos_v)
        pltpu.async_copy(os_hbm.at[pos_v], rows_v, sem).wait()
        pltpu.sync_copy(rows_v, r_hbm.at[pl.ds(base, bpw)])

    return k(osort, pos)


def _shared(x, wsg, wsu, wsdn):
    t, d = x.shape
    blk = 256

    def body(x_ref, g_ref, u_ref, dn_ref, o_ref):
        xb = x_ref[...]
        g = jnp.dot(xb, g_ref[...], preferred_element_type=jnp.float32)
        u = jnp.dot(xb, u_ref[...], preferred_element_type=jnp.float32)
        h = g * _sigmoid(g) * u
        o_ref[...] = jnp.dot(h, dn_ref[...], preferred_element_type=jnp.float32)

    return pl.pallas_call(
        body,
        grid=(t // blk,),
        in_specs=[
            pl.BlockSpec((blk, d), lambda i: (i, 0)),
            pl.BlockSpec(wsg.shape, lambda i: (0, 0)),
            pl.BlockSpec(wsu.shape, lambda i: (0, 0)),
            pl.BlockSpec(wsdn.shape, lambda i: (0, 0)),
        ],
        out_specs=pl.BlockSpec((blk, d), lambda i: (i, 0)),
        out_shape=jax.ShapeDtypeStruct((t, d), jnp.float32),
    )(x, wsg, wsu, wsdn)


def _add(a, b):
    t, d = a.shape
    blk = 512

    def body(a_ref, b_ref, o_ref):
        o_ref[...] = a_ref[...] + b_ref[...].astype(jnp.float32)

    return pl.pallas_call(
        body,
        grid=(t // blk,),
        in_specs=[
            pl.BlockSpec((blk, a.shape[1]), lambda i: (i, 0)),
            pl.BlockSpec((blk, a.shape[1]), lambda i: (i, 0)),
        ],
        out_specs=pl.BlockSpec((blk, d), lambda i: (i, 0)),
        out_shape=jax.ShapeDtypeStruct((t, d), jnp.float32),
    )(a, b)


def kernel(hidden_states, router_w, gate_proj, up_proj, down_proj,
           shared_gate, shared_up, shared_down):
    b, s, d = hidden_states.shape
    t = b * s
    num_e = router_w.shape[1]
    nt = t // TILE + num_e

    x = hidden_states.reshape(t, d)

    xs, scores_te, eid3, grk3, cnt3, meta = _router(x, router_w)
    eid = eid3.reshape(t)
    grk = grk3.reshape(t)
    cnt = cnt3.reshape(2 * num_e)

    y, pos = _dispatch(xs, eid, grk, cnt, nt)
    osort = _grouped(meta, y, gate_proj, up_proj, down_proj, nt)
    routed = _combine(osort, pos)

    shared = _shared(x, shared_gate, shared_up, shared_down)
    out = _add(shared, routed)
    return out, scores_te.T


# speedup vs baseline: 1.0417x; 1.0006x over previous
import dataclasses
import functools

import jax
import jax.numpy as jnp
from jax import lax
from jax.experimental import pallas as pl
from jax.experimental.pallas import tpu as pltpu
from jax.experimental.pallas import tpu_sc as plsc

TILE = 256
TOK_BLK = 1024
NW = 32
LANES = 16


def _sigmoid(x):
    return 1.0 / (1.0 + jnp.exp(-x))


def _sc_compiler_params():
    cp = pltpu.CompilerParams()
    if "needs_layout_passes" in pltpu.CompilerParams.__dataclass_fields__:
        cp = dataclasses.replace(cp, needs_layout_passes=False)
    return cp


def _router_body(x_ref, rw_ref, xs_ref, sc_ref, eid_ref, grk_ref, cnt_ref,
                 meta_ref, counts):
    i = pl.program_id(0)
    nblk = pl.num_programs(0)
    num_e = rw_ref.shape[1]

    @pl.when(i == 0)
    def _():
        counts[...] = jnp.zeros_like(counts)

    x = x_ref[...]
    logits = jnp.dot(x, rw_ref[...], preferred_element_type=jnp.float32)
    m = jnp.max(logits, axis=1, keepdims=True)
    iota_e = lax.broadcasted_iota(jnp.int32, logits.shape, 1)
    eid = jnp.min(jnp.where(logits == m, iota_e, num_e), axis=1,
                  keepdims=True)
    onehot = (iota_e == eid).astype(jnp.float32)
    s = _sigmoid(m)
    xs_ref[...] = (x * s).astype(xs_ref.dtype)

    sc_ref[...] = _sigmoid(jnp.where(onehot > 0, logits, -jnp.inf))

    ii = lax.broadcasted_iota(jnp.int32, (TOK_BLK, TOK_BLK), 0)
    jj = lax.broadcasted_iota(jnp.int32, (TOK_BLK, TOK_BLK), 1)
    tril = (ii > jj).astype(jnp.float32)

    prefix = jnp.dot(tril, onehot, preferred_element_type=jnp.float32)
    local = jnp.sum(prefix * onehot, axis=1, keepdims=True)
    carried = jnp.sum(onehot * counts[...], axis=1, keepdims=True)
    grank = local + carried
    counts[...] = counts[...] + jnp.sum(onehot, axis=0, keepdims=True)

    eid_ref[...] = eid.reshape(1, TOK_BLK, 1)
    grk_ref[...] = grank.astype(jnp.int32).reshape(1, TOK_BLK, 1)

    @pl.when(i == nblk - 1)
    def _():
        c16 = jnp.concatenate([counts[...], jnp.zeros_like(counts)], axis=1)
        cnt_ref[...] = c16.astype(jnp.int32).reshape(1, 1, 2 * num_e)

        pc16 = jnp.ceil(c16 * (1.0 / TILE)) * TILE
        u16i = lax.broadcasted_iota(jnp.int32, (16, 16), 0)
        u16j = lax.broadcasted_iota(jnp.int32, (16, 16), 1)
        u16 = (u16i <= u16j).astype(jnp.float32)
        incl = jnp.dot(jnp.broadcast_to(pc16, (8, 16)), u16,
                       preferred_element_type=jnp.float32)[0:1]
        total = jnp.max(incl, keepdims=True)
        last = total * (1.0 / TILE) - 1.0
        nt2 = meta_ref.shape[1]
        itile = lax.broadcasted_iota(jnp.int32, (nt2, 1), 0).astype(jnp.float32)
        ic_col = jnp.minimum(itile, last)
        start_col = ic_col * TILE
        cmp = (incl <= start_col).astype(jnp.float32)
        te_col = jnp.sum(cmp, axis=1, keepdims=True)
        ii2 = lax.broadcasted_iota(jnp.int32, (nt2, nt2), 0)
        jj2 = lax.broadcasted_iota(jnp.int32, (nt2, nt2), 1)
        ident2 = (ii2 == jj2).astype(jnp.float32)

        def dot_t2(a):
            return lax.dot_general(a, ident2, (((0,), (0,)), ((), ())),
                                   preferred_element_type=jnp.float32)

        meta = jnp.concatenate(
            [dot_t2(te_col), dot_t2(ic_col),
             jnp.zeros((meta_ref.shape[0] - 2, nt2), jnp.float32)], axis=0)
        meta_ref[...] = meta.astype(jnp.int32)


def _router(x, rw):
    t, d = x.shape
    nblk = t // TOK_BLK
    num_e = rw.shape[1]
    return pl.pallas_call(
        _router_body,
        grid=(nblk,),
        in_specs=[
            pl.BlockSpec((TOK_BLK, d), lambda i: (i, 0)),
            pl.BlockSpec((d, num_e), lambda i: (0, 0)),
        ],
        out_specs=[
            pl.BlockSpec((TOK_BLK, d), lambda i: (i, 0)),
            pl.BlockSpec((TOK_BLK, num_e), lambda i: (i, 0)),
            pl.BlockSpec((1, TOK_BLK, 1), lambda i: (i, 0, 0)),
            pl.BlockSpec((1, TOK_BLK, 1), lambda i: (i, 0, 0)),
            pl.BlockSpec((1, 1, 2 * num_e), lambda i: (0, 0, 0)),
            pl.BlockSpec((8, 2 * LANES), lambda i: (0, 0)),
        ],
        out_shape=[
            jax.ShapeDtypeStruct((t, d), jnp.float32),
            jax.ShapeDtypeStruct((t, num_e), jnp.float32),
            jax.ShapeDtypeStruct((nblk, TOK_BLK, 1), jnp.int32),
            jax.ShapeDtypeStruct((nblk, TOK_BLK, 1), jnp.int32),
            jax.ShapeDtypeStruct((1, 1, 2 * num_e), jnp.int32),
            jax.ShapeDtypeStruct((8, 2 * LANES), jnp.int32),
        ],
        scratch_shapes=[pltpu.VMEM((1, num_e), jnp.float32)],
    )(x, rw)


def _dispatch(xs, eid, grank, counts, nt):
    t, d = xs.shape
    bpw = t // NW
    num_e = 8
    mesh = plsc.VectorSubcoreMesh(core_axis_name="c", subcore_axis_name="s")

    @functools.partial(
        pl.kernel,
        mesh=mesh,
        compiler_params=_sc_compiler_params(),
        out_type=[
            jax.ShapeDtypeStruct((nt * TILE, d), jnp.float32),
            jax.ShapeDtypeStruct((t,), jnp.int32),
        ],
        scratch_types=[
            pltpu.VMEM((bpw,), jnp.int32),
            pltpu.VMEM((bpw,), jnp.int32),
            pltpu.VMEM((1, bpw), jnp.int32),
            pltpu.VMEM((LANES,), jnp.int32),
            pltpu.VMEM((LANES,), jnp.int32),
            pltpu.VMEM((bpw, d), jnp.float32),
        ],
    )
    def k(xs_hbm, eid_hbm, grk_hbm, cnt_hbm, y_hbm, pos_hbm,
          eid_v, grk_v, pos_v, cnt_v, off_v, rows_v):
        wid = lax.axis_index("s") * 2 + lax.axis_index("c")
        base = wid * bpw
        pltpu.sync_copy(eid_hbm.at[pl.ds(base, bpw)], eid_v)
        pltpu.sync_copy(grk_hbm.at[pl.ds(base, bpw)], grk_v)
        pltpu.sync_copy(cnt_hbm, cnt_v)
        c = cnt_v[...]
        pc = (c + (TILE - 1)) & (-TILE)
        incl = plsc.cumsum(pc)
        off_v[...] = incl - pc
        for j in range(bpw // LANES):
            ev = eid_v[pl.ds(LANES * j, LANES)]
            gv = grk_v[pl.ds(LANES * j, LANES)]
            ov = plsc.load_gather(off_v, [ev])
            pos_v[0, pl.ds(LANES * j, LANES)] = ov + gv
        pltpu.sync_copy(pos_v.at[0], pos_hbm.at[pl.ds(base, bpw)])
        pltpu.sync_copy(xs_hbm.at[pl.ds(base, bpw)], rows_v)
        pltpu.sync_copy(rows_v, y_hbm.at[pos_v.at[0]])

    return k(xs, eid, grank, counts)


def _grouped(meta, y, wg, wu, wdn, nt):
    _, d, f = wg.shape

    def body(meta_ref, y_ref, wg_ref, wu_ref, wdn_ref, o_ref):
        i = pl.program_id(0)

        @pl.when(i == meta_ref[1, i])
        def _():
            yb = y_ref[...]
            g = jnp.dot(yb, wg_ref[0], preferred_element_type=jnp.float32)
            u = jnp.dot(yb, wu_ref[0], preferred_element_type=jnp.float32)
            h = g * _sigmoid(g) * u
            o_ref[...] = jnp.dot(h, wdn_ref[0],
                                 preferred_element_type=jnp.float32)

    grid_spec = pltpu.PrefetchScalarGridSpec(
        num_scalar_prefetch=1,
        grid=(nt,),
        in_specs=[
            pl.BlockSpec((TILE, d), lambda i, m: (m[1, i], 0)),
            pl.BlockSpec((1, d, f), lambda i, m: (m[0, i], 0, 0)),
            pl.BlockSpec((1, d, f), lambda i, m: (m[0, i], 0, 0)),
            pl.BlockSpec((1, f, d), lambda i, m: (m[0, i], 0, 0)),
        ],
        out_specs=pl.BlockSpec((TILE, d), lambda i, m: (m[1, i], 0)),
    )
    return pl.pallas_call(
        body,
        grid_spec=grid_spec,
        out_shape=jax.ShapeDtypeStruct((nt * TILE, d), jnp.float32),
    )(meta, y, wg, wu, wdn)


def _combine(osort, pos):
    t = pos.shape[0]
    d = osort.shape[1]
    bpw = t // NW
    mesh = plsc.VectorSubcoreMesh(core_axis_name="c", subcore_axis_name="s")

    @functools.partial(
        pl.kernel,
        mesh=mesh,
        compiler_params=_sc_compiler_params(),
        out_type=jax.ShapeDtypeStruct((t, d), jnp.float32),
        scratch_types=[
            pltpu.VMEM((bpw,), jnp.int32),
            pltpu.VMEM((bpw, d), jnp.float32),
            pltpu.SemaphoreType.DMA,
        ],
    )
    def k(os_hbm, pos_hbm, r_hbm, pos_v, rows_v, sem):
        wid = lax.axis_index("s") * 2 + lax.axis_index("c")
        base = wid * bpw
        pltpu.sync_copy(pos_hbm.at[pl.ds(base, bpw)], pos_v)
        pltpu.async_copy(os_hbm.at[pos_v], rows_v, sem).wait()
        pltpu.sync_copy(rows_v, r_hbm.at[pl.ds(base, bpw)])

    return k(osort, pos)


def _shared(x, wsg, wsu, wsdn):
    t, d = x.shape
    blk = 256

    def body(x_ref, g_ref, u_ref, dn_ref, o_ref):
        xb = x_ref[...]
        g = jnp.dot(xb, g_ref[...], preferred_element_type=jnp.float32)
        u = jnp.dot(xb, u_ref[...], preferred_element_type=jnp.float32)
        h = g * _sigmoid(g) * u
        o_ref[...] = jnp.dot(h, dn_ref[...], preferred_element_type=jnp.float32)

    return pl.pallas_call(
        body,
        grid=(t // blk,),
        in_specs=[
            pl.BlockSpec((blk, d), lambda i: (i, 0)),
            pl.BlockSpec(wsg.shape, lambda i: (0, 0)),
            pl.BlockSpec(wsu.shape, lambda i: (0, 0)),
            pl.BlockSpec(wsdn.shape, lambda i: (0, 0)),
        ],
        out_specs=pl.BlockSpec((blk, d), lambda i: (i, 0)),
        out_shape=jax.ShapeDtypeStruct((t, d), jnp.float32),
    )(x, wsg, wsu, wsdn)


def _add(a, b):
    t, d = a.shape
    blk = 512

    def body(a_ref, b_ref, o_ref):
        o_ref[...] = a_ref[...] + b_ref[...].astype(jnp.float32)

    return pl.pallas_call(
        body,
        grid=(t // blk,),
        in_specs=[
            pl.BlockSpec((blk, a.shape[1]), lambda i: (i, 0)),
            pl.BlockSpec((blk, a.shape[1]), lambda i: (i, 0)),
        ],
        out_specs=pl.BlockSpec((blk, d), lambda i: (i, 0)),
        out_shape=jax.ShapeDtypeStruct((t, d), jnp.float32),
    )(a, b)


def kernel(hidden_states, router_w, gate_proj, up_proj, down_proj,
           shared_gate, shared_up, shared_down):
    b, s, d = hidden_states.shape
    t = b * s
    num_e = router_w.shape[1]
    nt = t // TILE + num_e

    x = hidden_states.reshape(t, d)

    xs, scores_te, eid3, grk3, cnt3, meta = _router(x, router_w)
    eid = eid3.reshape(t)
    grk = grk3.reshape(t)
    cnt = cnt3.reshape(2 * num_e)

    y, pos = _dispatch(xs, eid, grk, cnt, nt)
    osort = _grouped(meta, y, gate_proj, up_proj, down_proj, nt)
    routed = _combine(osort, pos)

    shared = _shared(x, shared_gate, shared_up, shared_down)
    out = _add(shared, routed)
    return out, scores_te.T


# speedup vs baseline: 1.0646x; 1.0220x over previous
import dataclasses
import functools

import jax
import jax.numpy as jnp
from jax import lax
from jax.experimental import pallas as pl
from jax.experimental.pallas import tpu as pltpu
from jax.experimental.pallas import tpu_sc as plsc

TILE = 256
TOK_BLK = 1024
NW = 32
LANES = 16


def _sigmoid(x):
    return 1.0 / (1.0 + jnp.exp(-x))


def _sc_compiler_params():
    cp = pltpu.CompilerParams()
    if "needs_layout_passes" in pltpu.CompilerParams.__dataclass_fields__:
        cp = dataclasses.replace(cp, needs_layout_passes=False)
    return cp


def _router_body(x_ref, rw_ref, xs_ref, sc_ref, eid_ref, grk_ref, cnt_ref,
                 meta_ref, counts):
    i = pl.program_id(0)
    nblk = pl.num_programs(0)
    num_e = rw_ref.shape[1]

    @pl.when(i == 0)
    def _():
        counts[...] = jnp.zeros_like(counts)

    x = x_ref[...]
    logits = jnp.dot(x, rw_ref[...], preferred_element_type=jnp.float32)
    m = jnp.max(logits, axis=1, keepdims=True)
    iota_e = lax.broadcasted_iota(jnp.int32, logits.shape, 1)
    eid = jnp.min(jnp.where(logits == m, iota_e, num_e), axis=1,
                  keepdims=True)
    onehot = (iota_e == eid).astype(jnp.float32)
    s = _sigmoid(m)
    xs_ref[...] = (x * s).astype(xs_ref.dtype)

    sc_ref[...] = _sigmoid(jnp.where(onehot > 0, logits, -jnp.inf))

    ii = lax.broadcasted_iota(jnp.int32, (TOK_BLK, TOK_BLK), 0)
    jj = lax.broadcasted_iota(jnp.int32, (TOK_BLK, TOK_BLK), 1)
    tril = (ii > jj).astype(jnp.float32)

    prefix = jnp.dot(tril, onehot, preferred_element_type=jnp.float32)
    local = jnp.sum(prefix * onehot, axis=1, keepdims=True)
    carried = jnp.sum(onehot * counts[...], axis=1, keepdims=True)
    grank = local + carried
    counts[...] = counts[...] + jnp.sum(onehot, axis=0, keepdims=True)

    eid_ref[...] = eid.reshape(1, TOK_BLK, 1)
    grk_ref[...] = grank.astype(jnp.int32).reshape(1, TOK_BLK, 1)

    @pl.when(i == nblk - 1)
    def _():
        c16 = jnp.concatenate([counts[...], jnp.zeros_like(counts)], axis=1)
        cnt_ref[...] = c16.astype(jnp.int32).reshape(1, 1, 2 * num_e)

        pc16 = jnp.ceil(c16 * (1.0 / TILE)) * TILE
        u16i = lax.broadcasted_iota(jnp.int32, (16, 16), 0)
        u16j = lax.broadcasted_iota(jnp.int32, (16, 16), 1)
        u16 = (u16i <= u16j).astype(jnp.float32)
        incl = jnp.dot(jnp.broadcast_to(pc16, (8, 16)), u16,
                       preferred_element_type=jnp.float32)[0:1]
        total = jnp.max(incl, keepdims=True)
        last = total * (1.0 / TILE) - 1.0
        nt2 = meta_ref.shape[1]
        itile = lax.broadcasted_iota(jnp.int32, (nt2, 1), 0).astype(jnp.float32)
        ic_col = jnp.minimum(itile, last)
        start_col = ic_col * TILE
        cmp = (incl <= start_col).astype(jnp.float32)
        te_col = jnp.sum(cmp, axis=1, keepdims=True)
        ii2 = lax.broadcasted_iota(jnp.int32, (nt2, nt2), 0)
        jj2 = lax.broadcasted_iota(jnp.int32, (nt2, nt2), 1)
        ident2 = (ii2 == jj2).astype(jnp.float32)

        def dot_t2(a):
            return lax.dot_general(a, ident2, (((0,), (0,)), ((), ())),
                                   preferred_element_type=jnp.float32)

        meta = jnp.concatenate(
            [dot_t2(te_col), dot_t2(ic_col),
             jnp.zeros((meta_ref.shape[0] - 2, nt2), jnp.float32)], axis=0)
        meta_ref[...] = meta.astype(jnp.int32)


def _router(x, rw):
    t, d = x.shape
    nblk = t // TOK_BLK
    num_e = rw.shape[1]
    return pl.pallas_call(
        _router_body,
        grid=(nblk,),
        in_specs=[
            pl.BlockSpec((TOK_BLK, d), lambda i: (i, 0)),
            pl.BlockSpec((d, num_e), lambda i: (0, 0)),
        ],
        out_specs=[
            pl.BlockSpec((TOK_BLK, d), lambda i: (i, 0)),
            pl.BlockSpec((TOK_BLK, num_e), lambda i: (i, 0)),
            pl.BlockSpec((1, TOK_BLK, 1), lambda i: (i, 0, 0)),
            pl.BlockSpec((1, TOK_BLK, 1), lambda i: (i, 0, 0)),
            pl.BlockSpec((1, 1, 2 * num_e), lambda i: (0, 0, 0)),
            pl.BlockSpec((8, 2 * LANES), lambda i: (0, 0)),
        ],
        out_shape=[
            jax.ShapeDtypeStruct((t, d), jnp.float32),
            jax.ShapeDtypeStruct((t, num_e), jnp.float32),
            jax.ShapeDtypeStruct((nblk, TOK_BLK, 1), jnp.int32),
            jax.ShapeDtypeStruct((nblk, TOK_BLK, 1), jnp.int32),
            jax.ShapeDtypeStruct((1, 1, 2 * num_e), jnp.int32),
            jax.ShapeDtypeStruct((8, 2 * LANES), jnp.int32),
        ],
        scratch_shapes=[pltpu.VMEM((1, num_e), jnp.float32)],
    )(x, rw)


def _dispatch(xs, eid, grank, counts, nt):
    t, d = xs.shape
    bpw = t // NW
    num_e = 8
    mesh = plsc.VectorSubcoreMesh(core_axis_name="c", subcore_axis_name="s")

    @functools.partial(
        pl.kernel,
        mesh=mesh,
        compiler_params=_sc_compiler_params(),
        out_type=[
            jax.ShapeDtypeStruct((nt * TILE, d), jnp.float32),
            jax.ShapeDtypeStruct((t,), jnp.int32),
        ],
        scratch_types=[
            pltpu.VMEM((bpw,), jnp.int32),
            pltpu.VMEM((bpw,), jnp.int32),
            pltpu.VMEM((1, bpw), jnp.int32),
            pltpu.VMEM((LANES,), jnp.int32),
            pltpu.VMEM((LANES,), jnp.int32),
            pltpu.VMEM((bpw, d), jnp.float32),
        ],
    )
    def k(xs_hbm, eid_hbm, grk_hbm, cnt_hbm, y_hbm, pos_hbm,
          eid_v, grk_v, pos_v, cnt_v, off_v, rows_v):
        wid = lax.axis_index("s") * 2 + lax.axis_index("c")
        base = wid * bpw
        pltpu.sync_copy(eid_hbm.at[pl.ds(base, bpw)], eid_v)
        pltpu.sync_copy(grk_hbm.at[pl.ds(base, bpw)], grk_v)
        pltpu.sync_copy(cnt_hbm, cnt_v)
        c = cnt_v[...]
        pc = (c + (TILE - 1)) & (-TILE)
        incl = plsc.cumsum(pc)
        off_v[...] = incl - pc
        for j in range(bpw // LANES):
            ev = eid_v[pl.ds(LANES * j, LANES)]
            gv = grk_v[pl.ds(LANES * j, LANES)]
            ov = plsc.load_gather(off_v, [ev])
            pos_v[0, pl.ds(LANES * j, LANES)] = ov + gv
        pltpu.sync_copy(pos_v.at[0], pos_hbm.at[pl.ds(base, bpw)])
        pltpu.sync_copy(xs_hbm.at[pl.ds(base, bpw)], rows_v)
        pltpu.sync_copy(rows_v, y_hbm.at[pos_v.at[0]])

    return k(xs, eid, grank, counts)


def _grouped(meta, y, wg, wu, wdn, nt):
    _, d, f = wg.shape

    def body(meta_ref, y_ref, wg_ref, wu_ref, wdn_ref, o_ref):
        i = pl.program_id(0)

        @pl.when(i == meta_ref[1, i])
        def _():
            yb = y_ref[...]
            g = jnp.dot(yb, wg_ref[0], preferred_element_type=jnp.float32)
            u = jnp.dot(yb, wu_ref[0], preferred_element_type=jnp.float32)
            h = g * _sigmoid(g) * u
            o_ref[...] = jnp.dot(h, wdn_ref[0],
                                 preferred_element_type=jnp.float32)

    grid_spec = pltpu.PrefetchScalarGridSpec(
        num_scalar_prefetch=1,
        grid=(nt,),
        in_specs=[
            pl.BlockSpec((TILE, d), lambda i, m: (m[1, i], 0)),
            pl.BlockSpec((1, d, f), lambda i, m: (m[0, i], 0, 0)),
            pl.BlockSpec((1, d, f), lambda i, m: (m[0, i], 0, 0)),
            pl.BlockSpec((1, f, d), lambda i, m: (m[0, i], 0, 0)),
        ],
        out_specs=pl.BlockSpec((TILE, d), lambda i, m: (m[1, i], 0)),
    )
    return pl.pallas_call(
        body,
        grid_spec=grid_spec,
        out_shape=jax.ShapeDtypeStruct((nt * TILE, d), jnp.float32),
    )(meta, y, wg, wu, wdn)


def _combine_add(osort, pos, shared):
    t = pos.shape[0]
    d = osort.shape[1]
    bpw = t // NW
    half = bpw // 2
    mesh = plsc.VectorSubcoreMesh(core_axis_name="c", subcore_axis_name="s")

    @functools.partial(
        pl.kernel,
        mesh=mesh,
        compiler_params=_sc_compiler_params(),
        out_type=jax.ShapeDtypeStruct((t, d), jnp.float32),
        scratch_types=[
            pltpu.VMEM((bpw,), jnp.int32),
            pltpu.VMEM((half, d), jnp.float32),
            pltpu.VMEM((half, d), jnp.float32),
            pltpu.SemaphoreType.DMA,
        ],
    )
    def k(os_hbm, pos_hbm, sh_hbm, o_hbm, pos_v, rows_v, sh_v, sem):
        wid = lax.axis_index("s") * 2 + lax.axis_index("c")
        base = wid * bpw
        pltpu.sync_copy(pos_hbm.at[pl.ds(base, bpw)], pos_v)
        for hf in range(2):
            hb = base + hf * half
            pltpu.async_copy(os_hbm.at[pos_v.at[pl.ds(hf * half, half)]],
                             rows_v, sem).wait()
            pltpu.sync_copy(sh_hbm.at[pl.ds(hb, half)], sh_v)

            @pl.loop(0, half)
            def _(r):
                for j in range(d // LANES):
                    sl = (r, pl.ds(LANES * j, LANES))
                    rows_v[sl] = rows_v[sl] + sh_v[sl]

            pltpu.sync_copy(rows_v, o_hbm.at[pl.ds(hb, half)])

    return k(osort, pos, shared)


def _shared(x, wsg, wsu, wsdn):
    t, d = x.shape
    blk = 256

    def body(x_ref, g_ref, u_ref, dn_ref, o_ref):
        xb = x_ref[...]
        g = jnp.dot(xb, g_ref[...], preferred_element_type=jnp.float32)
        u = jnp.dot(xb, u_ref[...], preferred_element_type=jnp.float32)
        h = g * _sigmoid(g) * u
        o_ref[...] = jnp.dot(h, dn_ref[...], preferred_element_type=jnp.float32)

    return pl.pallas_call(
        body,
        grid=(t // blk,),
        in_specs=[
            pl.BlockSpec((blk, d), lambda i: (i, 0)),
            pl.BlockSpec(wsg.shape, lambda i: (0, 0)),
            pl.BlockSpec(wsu.shape, lambda i: (0, 0)),
            pl.BlockSpec(wsdn.shape, lambda i: (0, 0)),
        ],
        out_specs=pl.BlockSpec((blk, d), lambda i: (i, 0)),
        out_shape=jax.ShapeDtypeStruct((t, d), jnp.float32),
    )(x, wsg, wsu, wsdn)


def _add(a, b):
    t, d = a.shape
    blk = 512

    def body(a_ref, b_ref, o_ref):
        o_ref[...] = a_ref[...] + b_ref[...].astype(jnp.float32)

    return pl.pallas_call(
        body,
        grid=(t // blk,),
        in_specs=[
            pl.BlockSpec((blk, a.shape[1]), lambda i: (i, 0)),
            pl.BlockSpec((blk, a.shape[1]), lambda i: (i, 0)),
        ],
        out_specs=pl.BlockSpec((blk, d), lambda i: (i, 0)),
        out_shape=jax.ShapeDtypeStruct((t, d), jnp.float32),
    )(a, b)


def kernel(hidden_states, router_w, gate_proj, up_proj, down_proj,
           shared_gate, shared_up, shared_down):
    b, s, d = hidden_states.shape
    t = b * s
    num_e = router_w.shape[1]
    nt = t // TILE + num_e

    x = hidden_states.reshape(t, d)

    xs, scores_te, eid3, grk3, cnt3, meta = _router(x, router_w)
    eid = eid3.reshape(t)
    grk = grk3.reshape(t)
    cnt = cnt3.reshape(2 * num_e)

    y, pos = _dispatch(xs, eid, grk, cnt, nt)
    osort = _grouped(meta, y, gate_proj, up_proj, down_proj, nt)

    shared = _shared(x, shared_gate, shared_up, shared_down)
    out = _combine_add(osort, pos, shared)
    return out, scores_te.T
